# baseline (speedup 1.0000x reference)
import jax
import jax.numpy as jnp
from jax.experimental import pallas as pl
from jax.experimental.pallas import tpu as pltpu

BLK = 64


def _mm_body(st_ref, en_ref, bi_ref, ei_ref, xs_ref, w_ref, o_ref):
    g = pl.program_id(0)
    start = st_ref[g]
    end = en_ref[g]
    base = bi_ref[g] * BLK
    pos = base + jax.lax.broadcasted_iota(jnp.int32, (BLK, 1), 0)
    mask = (pos >= start) & (pos < end)
    xm = jnp.where(mask, xs_ref[...], 0.0)
    contrib = jax.lax.dot_general(
        xm, w_ref[0], (((1,), (1,)), ((), ())),
        preferred_element_type=jnp.float32)

    @pl.when(start == base)
    def _init():
        o_ref[...] = contrib

    @pl.when(start != base)
    def _acc():
        o_ref[...] += contrib


def kernel(x, indices, control_vectors):
    T, H = x.shape
    E = control_vectors.shape[0]
    NB = T // BLK
    G = NB + E

    se, sort_idx = jax.lax.sort(
        (indices, jnp.arange(T, dtype=jnp.int32)), num_keys=1)
    pos = jnp.arange(T, dtype=jnp.int32)
    prev = jnp.concatenate([se[:1], se[:-1]])
    marker = (pos % BLK == 0) | (se != prev)
    cand = jnp.where(marker, pos, T)
    cand_sorted = jnp.sort(cand)
    starts = cand_sorted[:G].astype(jnp.int32)
    ends = cand_sorted[1:G + 1].astype(jnp.int32)
    wp = jnp.minimum(starts, T - 1)
    bids = wp // BLK
    eids = jnp.take(se, wp, axis=0)

    x_sorted = jnp.take(x, sort_idx, axis=0)

    grid_spec = pltpu.PrefetchScalarGridSpec(
        num_scalar_prefetch=4,
        grid=(G,),
        in_specs=[
            pl.BlockSpec((BLK, H), lambda g, st, en, bi, ei: (bi[g], 0)),
            pl.BlockSpec((1, H, H), lambda g, st, en, bi, ei: (ei[g], 0, 0)),
        ],
        out_specs=pl.BlockSpec((BLK, H), lambda g, st, en, bi, ei: (bi[g], 0)),
    )
    out_sorted = pl.pallas_call(
        _mm_body,
        grid_spec=grid_spec,
        out_shape=jax.ShapeDtypeStruct((T, H), jnp.float32),
        compiler_params=pltpu.CompilerParams(
            dimension_semantics=("arbitrary",)),
    )(starts, ends, bids, eids, x_sorted, control_vectors)

    return jnp.zeros_like(x).at[sort_idx].set(out_sorted)


# speedup vs baseline: 1.0941x; 1.0941x over previous
import jax
import jax.numpy as jnp
from jax.experimental import pallas as pl
from jax.experimental.pallas import tpu as pltpu

BLK = 64


def _body(s_ref, nb_ref, off_ref, end_ref, x_ref, w_ref, o_ref):
    e = pl.program_id(0)
    T = o_ref.shape[0]

    @pl.when(e == 0)
    def _init():
        o_ref[...] = jnp.zeros_like(o_ref)

    w = w_ref[0]
    off = off_ref[e]
    end = end_ref[e]
    s0 = s_ref[e]

    def loop(i, carry):
        s = pl.multiple_of(jnp.minimum(s0 + i * BLK, T - BLK), 8)
        rows = x_ref[pl.ds(s, BLK), :]
        pos = s + jax.lax.broadcasted_iota(jnp.int32, (BLK, 1), 0)
        mask = (pos >= off) & (pos < end)
        xm = jnp.where(mask, rows, 0.0)
        contrib = jax.lax.dot_general(
            xm, w, (((1,), (1,)), ((), ())),
            preferred_element_type=jnp.float32)
        o_ref[pl.ds(s, BLK), :] += contrib
        return carry

    jax.lax.fori_loop(0, nb_ref[e], loop, 0)


def kernel(x, indices, control_vectors):
    T, H = x.shape
    E = control_vectors.shape[0]

    se, sort_idx = jax.lax.sort(
        (indices, jnp.arange(T, dtype=jnp.int32)), num_keys=1)
    cnt = jnp.zeros((E,), jnp.int32).at[indices].add(1)
    end = jnp.cumsum(cnt)
    off = end - cnt
    s0 = (off // 8) * 8
    nblk = jnp.where(cnt > 0, (end - s0 + BLK - 1) // BLK, 0).astype(jnp.int32)
    off = off.astype(jnp.int32)
    end = end.astype(jnp.int32)
    s0 = s0.astype(jnp.int32)

    x_sorted = jnp.take(x, sort_idx, axis=0)

    grid_spec = pltpu.PrefetchScalarGridSpec(
        num_scalar_prefetch=4,
        grid=(E,),
        in_specs=[
            pl.BlockSpec((T, H), lambda e, *_: (0, 0)),
            pl.BlockSpec((1, H, H), lambda e, *_: (e, 0, 0)),
        ],
        out_specs=pl.BlockSpec((T, H), lambda e, *_: (0, 0)),
    )
    out_sorted = pl.pallas_call(
        _body,
        grid_spec=grid_spec,
        out_shape=jax.ShapeDtypeStruct((T, H), jnp.float32),
        compiler_params=pltpu.CompilerParams(
            dimension_semantics=("arbitrary",)),
    )(s0, nblk, off, end, x_sorted, control_vectors)

    return jnp.zeros_like(x).at[sort_idx].set(out_sorted)


# speedup vs baseline: 1.1367x; 1.0389x over previous
import jax
import jax.numpy as jnp
from jax.experimental import pallas as pl
from jax.experimental.pallas import tpu as pltpu

BLK = 64


def _body(s_ref, nb_ref, off_ref, end_ref, x_ref, wa_ref, wb_ref, o_ref):
    e = pl.program_id(0)
    T = o_ref.shape[0]
    HO = wa_ref.shape[1]

    @pl.when(e == 0)
    def _init():
        o_ref[...] = jnp.zeros_like(o_ref)

    off = off_ref[e]
    end = end_ref[e]
    s0 = s_ref[e]

    def loop(i, carry):
        s = pl.multiple_of(jnp.minimum(s0 + i * BLK, T - BLK), 8)
        rows = x_ref[pl.ds(s, BLK), :]
        pos = s + jax.lax.broadcasted_iota(jnp.int32, (BLK, 1), 0)
        mask = (pos >= off) & (pos < end)
        xm = jnp.where(mask, rows, 0.0)
        ca = jax.lax.dot_general(
            xm, wa_ref[0], (((1,), (1,)), ((), ())),
            preferred_element_type=jnp.float32)
        cb = jax.lax.dot_general(
            xm, wb_ref[0], (((1,), (1,)), ((), ())),
            preferred_element_type=jnp.float32)
        o_ref[pl.ds(s, BLK), :HO] += ca
        o_ref[pl.ds(s, BLK), HO:] += cb
        return carry

    jax.lax.fori_loop(0, nb_ref[e], loop, 0)


def kernel(x, indices, control_vectors):
    T, H = x.shape
    E = control_vectors.shape[0]

    se, sort_idx = jax.lax.sort(
        (indices, jnp.arange(T, dtype=jnp.int32)), num_keys=1)
    cnt = jnp.zeros((E,), jnp.int32).at[indices].add(1)
    end = jnp.cumsum(cnt)
    off = end - cnt
    s0 = (off // 8) * 8
    nblk = jnp.where(cnt > 0, (end - s0 + BLK - 1) // BLK, 0).astype(jnp.int32)
    off = off.astype(jnp.int32)
    end = end.astype(jnp.int32)
    s0 = s0.astype(jnp.int32)

    x_sorted = jnp.take(x, sort_idx, axis=0)

    grid_spec = pltpu.PrefetchScalarGridSpec(
        num_scalar_prefetch=4,
        grid=(E,),
        in_specs=[
            pl.BlockSpec((T, H), lambda e, *_: (0, 0)),
            pl.BlockSpec((1, H // 2, H), lambda e, *_: (e, 0, 0)),
            pl.BlockSpec((1, H // 2, H), lambda e, *_: (e, 1, 0)),
        ],
        out_specs=pl.BlockSpec((T, H), lambda e, *_: (0, 0)),
    )
    out_sorted = pl.pallas_call(
        _body,
        grid_spec=grid_spec,
        out_shape=jax.ShapeDtypeStruct((T, H), jnp.float32),
        compiler_params=pltpu.CompilerParams(
            dimension_semantics=("arbitrary",)),
    )(s0, nblk, off, end, x_sorted, control_vectors, control_vectors)

    return jnp.zeros_like(x).at[sort_idx].set(out_sorted)
